# Initial kernel scaffold; baseline (speedup 1.0000x reference)
import jax
import jax.numpy as jnp
from jax.experimental import pallas as pl


def kernel(x, edge_index, batch, W1, b1, g1, bt1, W2, b2, g2, bt2, W3, b3, M1, mb1, mg1, mbt1, M2, mb2, mg2, mbt2, M3, mb3):
    raise NotImplementedError("write your pallas kernel here")


# speedup vs baseline: 16.4993x; 16.4993x over previous
import functools

import jax
import jax.numpy as jnp
from jax import lax
from jax.experimental import pallas as pl
from jax.experimental.pallas import tpu as pltpu
from jax.experimental.pallas import tpu_sc as plsc

N = 10000
E = 320000
NUM_GRAPHS = 64
D_IN = 128
H = 32

NC = 2
NS = 16
NW = NC * NS
CHUNK = 128
NCHUNK = 79
EP = NCHUNK * CHUNK
E_PAD = NW * EP
PAD_E = E_PAD - E
NPAD = 10240
RPT = NPAD // NS
DEGW = 8

_mesh = plsc.VectorSubcoreMesh(core_axis_name="c", subcore_axis_name="s")
_sc_params = pltpu.CompilerParams(use_tc_tiling_on_sc=False)


@functools.partial(
    pl.kernel,
    mesh=_mesh,
    compiler_params=_sc_params,
    out_type=jax.ShapeDtypeStruct((NC * NPAD, DEGW), jnp.float32),
    scratch_types=[
        pltpu.VMEM((CHUNK,), jnp.int32),
        pltpu.VMEM((CHUNK, DEGW), jnp.float32),
        pltpu.VMEM_SHARED((NPAD, DEGW), jnp.float32),
    ],
)
def _deg_kernel(dst_hbm, ones_hbm, zeros_hbm, out_hbm, dst_v, ones_v, acc_sh):
    c = lax.axis_index("c")
    s = lax.axis_index("s")
    wid = c * NS + s
    r0 = s * RPT
    pltpu.sync_copy(ones_hbm, ones_v)
    pltpu.sync_copy(zeros_hbm.at[pl.ds(r0, RPT)], acc_sh.at[pl.ds(r0, RPT)])
    plsc.subcore_barrier()

    def body(j, carry):
        base = wid * EP + j * CHUNK
        pltpu.sync_copy(dst_hbm.at[pl.ds(base, CHUNK)], dst_v)
        pltpu.sync_copy(ones_v, acc_sh.at[dst_v], add=True)
        return carry

    lax.fori_loop(0, NCHUNK, body, 0)
    plsc.subcore_barrier()
    pltpu.sync_copy(acc_sh.at[pl.ds(r0, RPT)],
                    out_hbm.at[pl.ds(c * NPAD + r0, RPT)])


@functools.partial(
    pl.kernel,
    mesh=_mesh,
    compiler_params=_sc_params,
    out_type=jax.ShapeDtypeStruct((NC * NPAD, H), jnp.float32),
    scratch_types=[
        pltpu.VMEM((CHUNK,), jnp.int32),
        pltpu.VMEM((CHUNK,), jnp.int32),
        pltpu.VMEM((CHUNK, H), jnp.float32),
        pltpu.VMEM_SHARED((NPAD, H), jnp.float32),
    ],
)
def _edge_sum_kernel(hs_hbm, src_hbm, dst_hbm, zeros_hbm, out_hbm,
                     src_v, dst_v, rows_v, acc_sh):
    c = lax.axis_index("c")
    s = lax.axis_index("s")
    wid = c * NS + s
    r0 = s * RPT
    pltpu.sync_copy(zeros_hbm.at[pl.ds(r0, RPT)], acc_sh.at[pl.ds(r0, RPT)])
    plsc.subcore_barrier()

    def body(j, carry):
        base = wid * EP + j * CHUNK
        pltpu.sync_copy(src_hbm.at[pl.ds(base, CHUNK)], src_v)
        pltpu.sync_copy(dst_hbm.at[pl.ds(base, CHUNK)], dst_v)
        pltpu.sync_copy(hs_hbm.at[src_v], rows_v)
        pltpu.sync_copy(rows_v, acc_sh.at[dst_v], add=True)
        return carry

    lax.fori_loop(0, NCHUNK, body, 0)
    plsc.subcore_barrier()
    pltpu.sync_copy(acc_sh.at[pl.ds(r0, RPT)],
                    out_hbm.at[pl.ds(c * NPAD + r0, RPT)])


def _row_iota():
    return lax.broadcasted_iota(jnp.int32, (NPAD, 1), 0)


def _stage1_body(x_ref, w1_ref, degs_ref, hs1_ref, dinv_ref):
    deg = degs_ref[0:NPAD, 0:1] + degs_ref[NPAD:2 * NPAD, 0:1] + 1.0
    dinv = 1.0 / jnp.sqrt(deg)
    h = jnp.dot(x_ref[...], w1_ref[...], preferred_element_type=jnp.float32)
    hs1_ref[...] = h * dinv
    dinv_ref[...] = dinv


def _mmbf(a, w):
    return jnp.dot(a, w, preferred_element_type=jnp.float32)


def _mid_body(sp_ref, hs_ref, dinv_ref, b_ref, g_ref, bt_ref, w_ref, out_ref):
    dinv = dinv_ref[...]
    pre = (sp_ref[0:NPAD, :] + sp_ref[NPAD:2 * NPAD, :] + hs_ref[...]) * dinv \
        + b_ref[...]
    mask = _row_iota() < N
    zm = jnp.where(mask, pre, 0.0)
    mu = jnp.sum(zm, axis=0, keepdims=True) / N
    var = jnp.sum(jnp.where(mask, (pre - mu) ** 2, 0.0), axis=0,
                  keepdims=True) / N
    y = (pre - mu) / jnp.sqrt(var + 1e-5) * g_ref[...] + bt_ref[...]
    y = jnp.where(mask, jnp.maximum(y, 0.0), 0.0)
    out_ref[...] = _mmbf(y, w_ref[...]) * dinv


def _bn64(t, g, b):
    mu = jnp.mean(t, axis=0, keepdims=True)
    var = jnp.mean((t - mu) ** 2, axis=0, keepdims=True)
    return (t - mu) / jnp.sqrt(var + 1e-5) * g + b


def _final_body(sp_ref, hs_ref, dinv_ref, b3_ref, batch_ref,
                m1_ref, mb1_ref, mg1_ref, mbt1_ref,
                m2_ref, mb2_ref, mg2_ref, mbt2_ref,
                m3_ref, mb3_ref, out_ref):
    z = (sp_ref[0:NPAD, :] + sp_ref[NPAD:2 * NPAD, :] + hs_ref[...]) \
        * dinv_ref[...] + b3_ref[...]
    z = jnp.where(_row_iota() < N, z, 0.0)
    gid = lax.broadcasted_iota(jnp.int32, (NUM_GRAPHS, NPAD), 0)
    onehot = (batch_ref[...] == gid).astype(jnp.float32)
    zg = jnp.dot(onehot, z, preferred_element_type=jnp.float32,
                 precision="highest")
    m = _mmbf(zg, m1_ref[...]) + mb1_ref[...]
    m = jnp.maximum(_bn64(m, mg1_ref[...], mbt1_ref[...]), 0.0)
    m = _mmbf(m, m2_ref[...]) + mb2_ref[...]
    m = jnp.maximum(_bn64(m, mg2_ref[...], mbt2_ref[...]), 0.0)
    out_ref[...] = _mmbf(m, m3_ref[...]) + mb3_ref[...]


def _tc(body, out_shape, *args):
    return pl.pallas_call(
        body, out_shape=out_shape)(*args)


def kernel(x, edge_index, batch, W1, b1, g1, bt1, W2, b2, g2, bt2, W3, b3,
           M1, mb1, mg1, mbt1, M2, mb2, mg2, mbt2, M3, mb3):
    f32 = jnp.float32
    pad_idx = jnp.full((PAD_E,), N, jnp.int32)
    src_p = jnp.concatenate([edge_index[0], pad_idx])
    dst_p = jnp.concatenate([edge_index[1], pad_idx])
    x_p = jnp.pad(x, ((0, NPAD - N), (0, 0)))
    batch_p = jnp.pad(batch, (0, NPAD - N)).reshape(1, NPAD)
    zeros_h = jnp.zeros((NPAD, H), f32)
    zeros_d = jnp.zeros((NPAD, DEGW), f32)
    ones_d = jnp.ones((CHUNK, DEGW), f32)

    degp = _deg_kernel(dst_p, ones_d, zeros_d)

    hs1, dinv = _tc(_stage1_body,
                    (jax.ShapeDtypeStruct((NPAD, H), f32),
                     jax.ShapeDtypeStruct((NPAD, 1), f32)),
                    x_p, W1, degp)

    s1 = _edge_sum_kernel(hs1, src_p, dst_p, zeros_h)
    hs2 = _tc(_mid_body, jax.ShapeDtypeStruct((NPAD, H), f32),
              s1, hs1, dinv, b1.reshape(1, H), g1.reshape(1, H),
              bt1.reshape(1, H), W2)

    s2 = _edge_sum_kernel(hs2, src_p, dst_p, zeros_h)
    hs3 = _tc(_mid_body, jax.ShapeDtypeStruct((NPAD, H), f32),
              s2, hs2, dinv, b2.reshape(1, H), g2.reshape(1, H),
              bt2.reshape(1, H), W3)

    s3 = _edge_sum_kernel(hs3, src_p, dst_p, zeros_h)
    out = _tc(_final_body, jax.ShapeDtypeStruct((NUM_GRAPHS, 1), f32),
              s3, hs3, dinv, b3.reshape(1, H), batch_p,
              M1, mb1.reshape(1, H), mg1.reshape(1, H), mbt1.reshape(1, H),
              M2, mb2.reshape(1, H), mg2.reshape(1, H), mbt2.reshape(1, H),
              M3, mb3.reshape(1, 1))
    return out
